# baseline (speedup 1.0000x reference)
import functools

import jax
import jax.numpy as jnp
from jax import lax
from jax.experimental import pallas as pl
from jax.experimental.pallas import tpu as pltpu
from jax.experimental.pallas import tpu_sc as plsc

B = 16384
F = 26
V = 100000
D = 16
H0, H1, H2 = 256, 128, 64
FD = F * D

NC, NS = 2, 16
NW = NC * NS
N = B * F
PER_W = N // NW
CH = 3328
NCH = PER_W // CH


def _sc_gather(idx, emb_flat, lin_flat):
    mesh = plsc.VectorSubcoreMesh(core_axis_name="c", subcore_axis_name="s")

    idx_e, idx_l = idx

    @functools.partial(
        pl.kernel,
        out_type=(
            jax.ShapeDtypeStruct((N, D), jnp.float32),
            jax.ShapeDtypeStruct((N,), jnp.float32),
        ),
        mesh=mesh,
        compiler_params=pltpu.CompilerParams(use_tc_tiling_on_sc=False),
        scratch_types=[
            pltpu.VMEM((PER_W,), jnp.int32),
            pltpu.VMEM((PER_W,), jnp.int32),
            pltpu.VMEM((CH, D), jnp.float32),
            pltpu.VMEM((PER_W,), jnp.float32),
            pltpu.SemaphoreType.DMA,
            pltpu.SemaphoreType.DMA,
        ],
    )
    def k(idxe_hbm, idxl_hbm, emb_hbm, lin_hbm, rows_out, lin_out,
          idxe_v, idxl_v, rows_v, lin_v, sem, sem2):
        wid = lax.axis_index("s") * NC + lax.axis_index("c")
        base = wid * PER_W
        pltpu.sync_copy(idxe_hbm.at[pl.ds(base, PER_W)], idxe_v)
        pltpu.sync_copy(idxl_hbm.at[pl.ds(base, PER_W)], idxl_v)
        lin_cp = pltpu.async_copy(lin_hbm.at[idxl_v], lin_v, sem2)
        for c in range(NCH):
            pltpu.async_copy(
                emb_hbm.at[idxe_v.at[pl.ds(c * CH, CH)]], rows_v, sem
            ).wait()
            pltpu.sync_copy(rows_v, rows_out.at[pl.ds(base + c * CH, CH)])
        lin_cp.wait()
        pltpu.sync_copy(lin_v, lin_out.at[pl.ds(base, PER_W)])

    return k(idx_e, idx_l, emb_flat, lin_flat)


VB = 8192
NVB = 13
VPG = NVB * VB
NG = 4
NROW16 = NG * VPG * 8


def _flatten_body(embT_ref, out_ref):
    out_ref[...] = embT_ref[...].T


def _tc_flatten(embT2):
    return pl.pallas_call(
        _flatten_body,
        grid=(NG, NVB),
        in_specs=[pl.BlockSpec((128, VB), lambda g, j: (g, j))],
        out_specs=pl.BlockSpec((VB, 128), lambda g, j: (g * NVB + j, 0)),
        out_shape=jax.ShapeDtypeStruct((NG * VPG, 128), jnp.float32),
    )(embT2)


BM = 1024


def _mlp_body(feat, linv, bias, w0, b0, w1, b1, w2, b2, w3, b3, out):
    x = feat[...]
    h = jnp.maximum(jnp.dot(x, w0[...], preferred_element_type=jnp.float32)
                    + b0[...], 0.0)
    h = jnp.maximum(jnp.dot(h, w1[...], preferred_element_type=jnp.float32)
                    + b1[...], 0.0)
    h = jnp.maximum(jnp.dot(h, w2[...], preferred_element_type=jnp.float32)
                    + b2[...], 0.0)
    o = jnp.dot(h, w3[...], preferred_element_type=jnp.float32) + b3[...]
    wide = jnp.sum(linv[...], axis=1, keepdims=True) + bias[...]
    out[...] = jax.nn.sigmoid(o + wide)


def _tc_mlp(feat, linv, bias, W0, b0, W1, b1, W2, b2, W3, b3):
    grid = (B // BM,)
    const = lambda i: (0, 0)
    return pl.pallas_call(
        _mlp_body,
        grid=grid,
        in_specs=[
            pl.BlockSpec((BM, FD), lambda i: (i, 0)),
            pl.BlockSpec((BM, F), lambda i: (i, 0)),
            pl.BlockSpec((1, 1), const),
            pl.BlockSpec((FD, H0), const),
            pl.BlockSpec((1, H0), const),
            pl.BlockSpec((H0, H1), const),
            pl.BlockSpec((1, H1), const),
            pl.BlockSpec((H1, H2), const),
            pl.BlockSpec((1, H2), const),
            pl.BlockSpec((H2, 1), const),
            pl.BlockSpec((1, 1), const),
        ],
        out_specs=pl.BlockSpec((BM, 1), lambda i: (i, 0)),
        out_shape=jax.ShapeDtypeStruct((B, 1), jnp.float32),
    )(feat, linv, bias, W0, b0, W1, b1, W2, b2, W3, b3)


def kernel(x, lin_tables, emb_tables, bias, W0, b0, W1, b1, W2, b2, W3, b3):
    xi = x.astype(jnp.int32)
    f_rng = jnp.arange(F, dtype=jnp.int32)
    idx_e = ((((f_rng // 8) * NVB)[None, :] + xi // VB) * (VB * 8)
             + (xi % VB) * 8 + (f_rng % 8)[None, :]).reshape(N)
    idx_l = (xi + (f_rng * V)[None, :]).reshape(N)
    embT = jnp.transpose(emb_tables, (0, 2, 1))
    embT2 = embT.reshape(F * D, V)
    emb_flat = _tc_flatten(embT2).reshape(NROW16, D)
    lin_flat = lin_tables.reshape(F * V)
    rows, linv = _sc_gather((idx_e, idx_l), emb_flat, lin_flat)
    return rows[:B, 0] + linv[:B]
    feat = rows.reshape(B, FD)
    linv = linv.reshape(B, F)
    out = _tc_mlp(feat, linv, bias.reshape(1, 1), W0, b0.reshape(1, H0),
                  W1, b1.reshape(1, H1), W2, b2.reshape(1, H2),
                  W3, b3.reshape(1, 1))
    return out.reshape(B)


# speedup vs baseline: 1.1704x; 1.1704x over previous
import functools

import jax
import jax.numpy as jnp
from jax import lax
from jax.experimental import pallas as pl
from jax.experimental.pallas import tpu as pltpu
from jax.experimental.pallas import tpu_sc as plsc

B = 16384
F = 26
V = 100000
D = 16
H0, H1, H2 = 256, 128, 64
FD = F * D

NC, NS = 2, 16
NW = NC * NS
N = B * F
PER_W = N // NW
CH = 3328
NCH = PER_W // CH


def _sc_gather(idx, emb_flat, lin_flat):
    mesh = plsc.VectorSubcoreMesh(core_axis_name="c", subcore_axis_name="s")

    idx_e, idx_l = idx

    @functools.partial(
        pl.kernel,
        out_type=(
            jax.ShapeDtypeStruct((N, D), jnp.float32),
            jax.ShapeDtypeStruct((N,), jnp.float32),
        ),
        mesh=mesh,
        compiler_params=pltpu.CompilerParams(use_tc_tiling_on_sc=False),
        scratch_types=[
            pltpu.VMEM((PER_W,), jnp.int32),
            pltpu.VMEM((PER_W,), jnp.int32),
            pltpu.VMEM((CH, D), jnp.float32),
            pltpu.VMEM((PER_W,), jnp.float32),
            pltpu.SemaphoreType.DMA,
            pltpu.SemaphoreType.DMA,
        ],
    )
    def k(idxe_hbm, idxl_hbm, emb_hbm, lin_hbm, rows_out, lin_out,
          idxe_v, idxl_v, rows_v, lin_v, sem, sem2):
        wid = lax.axis_index("s") * NC + lax.axis_index("c")
        base = wid * PER_W
        pltpu.sync_copy(idxe_hbm.at[pl.ds(base, PER_W)], idxe_v)
        pltpu.sync_copy(idxl_hbm.at[pl.ds(base, PER_W)], idxl_v)
        lin_cp = pltpu.async_copy(lin_hbm.at[idxl_v], lin_v, sem2)
        for c in range(NCH):
            pltpu.async_copy(
                emb_hbm.at[idxe_v.at[pl.ds(c * CH, CH)]], rows_v, sem
            ).wait()
            pltpu.sync_copy(rows_v, rows_out.at[pl.ds(base + c * CH, CH)])
        lin_cp.wait()
        pltpu.sync_copy(lin_v, lin_out.at[pl.ds(base, PER_W)])

    return k(idx_e, idx_l, emb_flat, lin_flat)


VB = 8192
NVB = 13
VPG = NVB * VB
NG = 4
NROW16 = NG * VPG * 8


def _flatten_body(embT_ref, out_ref):
    out_ref[...] = embT_ref[...].T


def _tc_flatten(embT2):
    return pl.pallas_call(
        _flatten_body,
        grid=(NG, NVB),
        in_specs=[pl.BlockSpec((128, VB), lambda g, j: (g, j))],
        out_specs=pl.BlockSpec((VB, 128), lambda g, j: (g * NVB + j, 0)),
        out_shape=jax.ShapeDtypeStruct((NG * VPG, 128), jnp.float32),
    )(embT2)


BM = 1024


def _mlp_body(feat, linv, bias, w0, b0, w1, b1, w2, b2, w3, b3, out):
    x = feat[...]
    h = jnp.maximum(jnp.dot(x, w0[...], preferred_element_type=jnp.float32)
                    + b0[...], 0.0)
    h = jnp.maximum(jnp.dot(h, w1[...], preferred_element_type=jnp.float32)
                    + b1[...], 0.0)
    h = jnp.maximum(jnp.dot(h, w2[...], preferred_element_type=jnp.float32)
                    + b2[...], 0.0)
    o = jnp.dot(h, w3[...], preferred_element_type=jnp.float32) + b3[...]
    wide = jnp.sum(linv[...], axis=1, keepdims=True) + bias[...]
    out[...] = jax.nn.sigmoid(o + wide)


def _tc_mlp(feat, linv, bias, W0, b0, W1, b1, W2, b2, W3, b3):
    grid = (B // BM,)
    const = lambda i: (0, 0)
    return pl.pallas_call(
        _mlp_body,
        grid=grid,
        in_specs=[
            pl.BlockSpec((BM, FD), lambda i: (i, 0)),
            pl.BlockSpec((BM, F), lambda i: (i, 0)),
            pl.BlockSpec((1, 1), const),
            pl.BlockSpec((FD, H0), const),
            pl.BlockSpec((1, H0), const),
            pl.BlockSpec((H0, H1), const),
            pl.BlockSpec((1, H1), const),
            pl.BlockSpec((H1, H2), const),
            pl.BlockSpec((1, H2), const),
            pl.BlockSpec((H2, 1), const),
            pl.BlockSpec((1, 1), const),
        ],
        out_specs=pl.BlockSpec((BM, 1), lambda i: (i, 0)),
        out_shape=jax.ShapeDtypeStruct((B, 1), jnp.float32),
    )(feat, linv, bias, W0, b0, W1, b1, W2, b2, W3, b3)


def kernel(x, lin_tables, emb_tables, bias, W0, b0, W1, b1, W2, b2, W3, b3):
    xi = x.astype(jnp.int32)
    f_rng = jnp.arange(F, dtype=jnp.int32)
    idx_e = ((((f_rng // 8) * NVB)[None, :] + xi // VB) * (VB * 8)
             + (xi % VB) * 8 + (f_rng % 8)[None, :]).reshape(N)
    idx_l = (xi + (f_rng * V)[None, :]).reshape(N)
    embT = jnp.transpose(emb_tables, (0, 2, 1))
    embT2 = embT.reshape(F * D, V)
    emb_flat = _tc_flatten(embT2).reshape(NROW16, D)
    lin_flat = lin_tables.reshape(F * V)
    rows, linv = _sc_gather((idx_e, idx_l), emb_flat, lin_flat)
    feat = rows.reshape(B, FD)
    linv = linv.reshape(B, F)
    out = _tc_mlp(feat, linv, bias.reshape(1, 1), W0, b0.reshape(1, H0),
                  W1, b1.reshape(1, H1), W2, b2.reshape(1, H2),
                  W3, b3.reshape(1, 1))
    return out.reshape(B)


# speedup vs baseline: 1.1793x; 1.0076x over previous
import functools

import jax
import jax.numpy as jnp
from jax import lax
from jax.experimental import pallas as pl
from jax.experimental.pallas import tpu as pltpu
from jax.experimental.pallas import tpu_sc as plsc

B = 16384
F = 26
V = 100000
D = 16
H0, H1, H2 = 256, 128, 64
FD = F * D

NC, NS = 2, 16
NW = NC * NS
N = B * F
PER_W = N // NW
CH = 1664
NCH = PER_W // CH


def _sc_gather(idx, emb_flat, lin_flat):
    mesh = plsc.VectorSubcoreMesh(core_axis_name="c", subcore_axis_name="s")

    idx_e, idx_l = idx

    @functools.partial(
        pl.kernel,
        out_type=(
            jax.ShapeDtypeStruct((N, D), jnp.float32),
            jax.ShapeDtypeStruct((N,), jnp.float32),
        ),
        mesh=mesh,
        compiler_params=pltpu.CompilerParams(use_tc_tiling_on_sc=False),
        scratch_types=[
            pltpu.VMEM((PER_W,), jnp.int32),
            pltpu.VMEM((PER_W,), jnp.int32),
            pltpu.VMEM((2, CH, D), jnp.float32),
            pltpu.VMEM((PER_W,), jnp.float32),
            pltpu.SemaphoreType.DMA,
            pltpu.SemaphoreType.DMA,
            pltpu.SemaphoreType.DMA,
        ],
    )
    def k(idxe_hbm, idxl_hbm, emb_hbm, lin_hbm, rows_out, lin_out,
          idxe_v, idxl_v, rows_v, lin_v, sem0, sem1, sem2):
        wid = lax.axis_index("s") * NC + lax.axis_index("c")
        base = wid * PER_W
        pltpu.sync_copy(idxe_hbm.at[pl.ds(base, PER_W)], idxe_v)
        pltpu.sync_copy(idxl_hbm.at[pl.ds(base, PER_W)], idxl_v)
        lin_cp = pltpu.async_copy(lin_hbm.at[idxl_v], lin_v, sem2)
        sems = (sem0, sem1)
        cps = [None, None]
        cps[0] = pltpu.async_copy(
            emb_hbm.at[idxe_v.at[pl.ds(0, CH)]], rows_v.at[0], sem0)
        for c in range(1, NCH + 1):
            if c < NCH:
                cps[c % 2] = pltpu.async_copy(
                    emb_hbm.at[idxe_v.at[pl.ds(c * CH, CH)]],
                    rows_v.at[c % 2], sems[c % 2])
            cps[(c - 1) % 2].wait()
            pltpu.sync_copy(rows_v.at[(c - 1) % 2],
                            rows_out.at[pl.ds(base + (c - 1) * CH, CH)])
        lin_cp.wait()
        pltpu.sync_copy(lin_v, lin_out.at[pl.ds(base, PER_W)])

    return k(idx_e, idx_l, emb_flat, lin_flat)


VB = 8192
NVB = 13
VPG = NVB * VB
NG = 4
NROW16 = NG * VPG * 8


def _flatten_body(embT_ref, out_ref):
    out_ref[...] = embT_ref[...].T


def _tc_flatten(embT2):
    return pl.pallas_call(
        _flatten_body,
        grid=(NG, NVB),
        in_specs=[pl.BlockSpec((128, VB), lambda g, j: (g, j))],
        out_specs=pl.BlockSpec((VB, 128), lambda g, j: (g * NVB + j, 0)),
        out_shape=jax.ShapeDtypeStruct((NG * VPG, 128), jnp.float32),
    )(embT2)


BM = 1024


def _mlp_body(feat, linv, bias, w0, b0, w1, b1, w2, b2, w3, b3, out):
    x = feat[...]
    h = jnp.maximum(jnp.dot(x, w0[...], preferred_element_type=jnp.float32)
                    + b0[...], 0.0)
    h = jnp.maximum(jnp.dot(h, w1[...], preferred_element_type=jnp.float32)
                    + b1[...], 0.0)
    h = jnp.maximum(jnp.dot(h, w2[...], preferred_element_type=jnp.float32)
                    + b2[...], 0.0)
    o = jnp.dot(h, w3[...], preferred_element_type=jnp.float32) + b3[...]
    wide = jnp.sum(linv[...], axis=1, keepdims=True) + bias[...]
    out[...] = jax.nn.sigmoid(o + wide)


def _tc_mlp(feat, linv, bias, W0, b0, W1, b1, W2, b2, W3, b3):
    grid = (B // BM,)
    const = lambda i: (0, 0)
    return pl.pallas_call(
        _mlp_body,
        grid=grid,
        in_specs=[
            pl.BlockSpec((BM, FD), lambda i: (i, 0)),
            pl.BlockSpec((BM, F), lambda i: (i, 0)),
            pl.BlockSpec((1, 1), const),
            pl.BlockSpec((FD, H0), const),
            pl.BlockSpec((1, H0), const),
            pl.BlockSpec((H0, H1), const),
            pl.BlockSpec((1, H1), const),
            pl.BlockSpec((H1, H2), const),
            pl.BlockSpec((1, H2), const),
            pl.BlockSpec((H2, 1), const),
            pl.BlockSpec((1, 1), const),
        ],
        out_specs=pl.BlockSpec((BM, 1), lambda i: (i, 0)),
        out_shape=jax.ShapeDtypeStruct((B, 1), jnp.float32),
    )(feat, linv, bias, W0, b0, W1, b1, W2, b2, W3, b3)


def kernel(x, lin_tables, emb_tables, bias, W0, b0, W1, b1, W2, b2, W3, b3):
    xi = x.astype(jnp.int32)
    f_rng = jnp.arange(F, dtype=jnp.int32)
    idx_e = ((((f_rng // 8) * NVB)[None, :] + xi // VB) * (VB * 8)
             + (xi % VB) * 8 + (f_rng % 8)[None, :]).reshape(N)
    idx_l = (xi + (f_rng * V)[None, :]).reshape(N)
    embT = jnp.transpose(emb_tables, (0, 2, 1))
    embT2 = embT.reshape(F * D, V)
    emb_flat = _tc_flatten(embT2).reshape(NROW16, D)
    lin_flat = lin_tables.reshape(F * V)
    rows, linv = _sc_gather((idx_e, idx_l), emb_flat, lin_flat)
    feat = rows.reshape(B, FD)
    linv = linv.reshape(B, F)
    out = _tc_mlp(feat, linv, bias.reshape(1, 1), W0, b0.reshape(1, H0),
                  W1, b1.reshape(1, H1), W2, b2.reshape(1, H2),
                  W3, b3.reshape(1, 1))
    return out.reshape(B)
